# Initial kernel scaffold; baseline (speedup 1.0000x reference)
import jax
import jax.numpy as jnp
from jax.experimental import pallas as pl


def kernel(mesh2grid_edge_features, grid_node_features, mesh_node_features, edge_index, We1, be1, We2, be2, ge_scale, ge_bias, Wn1, bn1, Wn2, bn2, gn_scale, gn_bias):
    raise NotImplementedError("write your pallas kernel here")


# speedup vs baseline: 2.1041x; 2.1041x over previous
import functools

import jax
import jax.numpy as jnp
from jax import lax
from jax.experimental import pallas as pl
from jax.experimental.pallas import tpu as pltpu
from jax.experimental.pallas import tpu_sc as plsc


def _ln(h, scale, bias):
    m = jnp.mean(h, axis=-1, keepdims=True)
    v = jnp.mean((h - m) * (h - m), axis=-1, keepdims=True)
    return (h - m) * lax.rsqrt(v + 1e-5) * scale + bias


def _preproj_body(mesh_ref, grid_ref, w1s_ref, w1d_ref, b1_ref, ps_ref, pd_ref):
    ps_ref[...] = jnp.dot(mesh_ref[...], w1s_ref[...],
                          preferred_element_type=jnp.float32)
    pd_ref[...] = jnp.dot(grid_ref[...], w1d_ref[...],
                          preferred_element_type=jnp.float32) + b1_ref[...]


def _tc_preproj(mesh_f, grid_f, w1s, w1d, b1):
    n, h = mesh_f.shape
    tn = 1000
    rows = pl.BlockSpec((tn, h), lambda i: (i, 0))
    full = pl.BlockSpec((h, h), lambda i: (0, 0))
    vec = pl.BlockSpec((1, h), lambda i: (0, 0))
    return pl.pallas_call(
        _preproj_body,
        grid=(n // tn,),
        in_specs=[rows, rows, full, full, vec],
        out_specs=[rows, rows],
        out_shape=[jax.ShapeDtypeStruct((n, h), jnp.float32)] * 2,
    )(mesh_f, grid_f, w1s, w1d, b1.reshape(1, h))


def _edge_body(e_ref, ga_ref, gb_ref, w1e_ref, w2_ref, b2_ref, s_ref, b_ref,
               out_ref):
    e = e_ref[...]
    h = jnp.dot(e, w1e_ref[...], preferred_element_type=jnp.float32)
    h = h + ga_ref[...] + gb_ref[...]
    h = h * jax.nn.sigmoid(h)
    h = jnp.dot(h, w2_ref[...], preferred_element_type=jnp.float32) + b2_ref[...]
    out_ref[...] = e + _ln(h, s_ref[...], b_ref[...])


def _tc_edge(e, ga, gb, w1e, w2, b2, sc, bi):
    ne, h = e.shape
    te = 800
    rows = pl.BlockSpec((te, h), lambda i: (i, 0))
    full = pl.BlockSpec((h, h), lambda i: (0, 0))
    vec = pl.BlockSpec((1, h), lambda i: (0, 0))
    return pl.pallas_call(
        _edge_body,
        grid=(ne // te,),
        in_specs=[rows, rows, rows, full, full, vec, vec, vec],
        out_specs=rows,
        out_shape=jax.ShapeDtypeStruct((ne, h), jnp.float32),
    )(e, ga, gb, w1e, w2, b2.reshape(1, h), sc.reshape(1, h), bi.reshape(1, h))


def _node_body(n_ref, agg_ref, w1a_ref, w1b_ref, b1_ref, w2_ref, b2_ref,
               s_ref, b_ref, out_ref):
    nd = n_ref[...]
    h = jnp.dot(nd, w1a_ref[...], preferred_element_type=jnp.float32)
    h = h + jnp.dot(agg_ref[...], w1b_ref[...],
                    preferred_element_type=jnp.float32) + b1_ref[...]
    h = h * jax.nn.sigmoid(h)
    h = jnp.dot(h, w2_ref[...], preferred_element_type=jnp.float32) + b2_ref[...]
    out_ref[...] = nd + _ln(h, s_ref[...], b_ref[...])


def _tc_node(node_f, agg, w1a, w1b, b1, w2, b2, sc, bi):
    n, h = node_f.shape
    tn = 1000
    rows = pl.BlockSpec((tn, h), lambda i: (i, 0))
    full = pl.BlockSpec((h, h), lambda i: (0, 0))
    vec = pl.BlockSpec((1, h), lambda i: (0, 0))
    return pl.pallas_call(
        _node_body,
        grid=(n // tn,),
        in_specs=[rows, rows, full, full, vec, full, vec, vec, vec],
        out_specs=rows,
        out_shape=jax.ShapeDtypeStruct((n, h), jnp.float32),
    )(node_f, agg, w1a, w1b, b1.reshape(1, h), w2, b2.reshape(1, h),
      sc.reshape(1, h), bi.reshape(1, h))


def _sc_gather(psrc, pdst, src_idx, dst_idx):
    n, h = psrc.shape
    e = src_idx.shape[0]
    nw = 32
    per_w = e // nw
    c = 40
    nch = per_w // c
    mesh = plsc.VectorSubcoreMesh(core_axis_name="c", subcore_axis_name="s")

    @functools.partial(
        pl.kernel, mesh=mesh,
        out_type=[jax.ShapeDtypeStruct((e, h), jnp.float32),
                  jax.ShapeDtypeStruct((e, h), jnp.float32)],
        scratch_types=[
            pltpu.VMEM((c,), jnp.int32), pltpu.VMEM((c,), jnp.int32),
            pltpu.VMEM((c, h), jnp.float32), pltpu.VMEM((c, h), jnp.float32),
            pltpu.SemaphoreType.DMA, pltpu.SemaphoreType.DMA,
        ])
    def k(ps_hbm, pd_hbm, si_hbm, di_hbm, ga_hbm, gb_hbm,
          idxs, idxd, bufa, bufb, sema, semb):
        wid = lax.axis_index("s") * 2 + lax.axis_index("c")

        def body(j, carry):
            base = wid * per_w + j * c
            pltpu.sync_copy(si_hbm.at[pl.ds(base, c)], idxs)
            pltpu.sync_copy(di_hbm.at[pl.ds(base, c)], idxd)
            ca = pltpu.async_copy(ps_hbm.at[idxs], bufa, sema)
            cb = pltpu.async_copy(pd_hbm.at[idxd], bufb, semb)
            ca.wait()
            cb.wait()
            pltpu.sync_copy(bufa, ga_hbm.at[pl.ds(base, c)])
            pltpu.sync_copy(bufb, gb_hbm.at[pl.ds(base, c)])
            return carry

        lax.fori_loop(0, nch, body, 0)

    return k(psrc, pdst, src_idx, dst_idx)


def _sc_scatter(eprime, dst_idx, zrows):
    e, h = eprime.shape
    n = zrows.shape[0]
    cb = 128
    nblk = h // cb // 2
    per_tile = e // 16
    cs = 80
    nch = per_tile // cs
    rc = 400
    nrc = n // rc
    nrit = (nrc + 15) // 16
    mesh = plsc.VectorSubcoreMesh(core_axis_name="c", subcore_axis_name="s")

    @functools.partial(
        pl.kernel, mesh=mesh,
        out_type=jax.ShapeDtypeStruct((n, h), jnp.float32),
        scratch_types=[
            pltpu.VMEM((cs,), jnp.int32),
            pltpu.VMEM((cs, cb), jnp.float32),
            pltpu.VMEM_SHARED((n, cb), jnp.float32),
        ])
    def k(ep_hbm, di_hbm, z_hbm, agg_hbm, idxv, ebuf, acc):
        c = lax.axis_index("c")
        s = lax.axis_index("s")
        for bb in range(nblk):
            col = (bb * 2 + c) * cb

            def zbody(j, carry):
                idx = j * 16 + s

                @pl.when(idx < nrc)
                def _():
                    rb = idx * rc
                    pltpu.sync_copy(z_hbm.at[pl.ds(rb, rc)],
                                    acc.at[pl.ds(rb, rc)])
                return carry

            lax.fori_loop(0, nrit, zbody, 0)
            plsc.subcore_barrier()

            def body(j, carry):
                base = s * per_tile + j * cs
                pltpu.sync_copy(di_hbm.at[pl.ds(base, cs)], idxv)
                pltpu.sync_copy(ep_hbm.at[pl.ds(base, cs), pl.ds(col, cb)],
                                ebuf)
                pltpu.sync_copy(ebuf, acc.at[idxv], add=True)
                return carry

            lax.fori_loop(0, nch, body, 0)
            plsc.subcore_barrier()

            def wbody(j, carry):
                idx = j * 16 + s

                @pl.when(idx < nrc)
                def _():
                    rb = idx * rc
                    pltpu.sync_copy(acc.at[pl.ds(rb, rc)],
                                    agg_hbm.at[pl.ds(rb, rc), pl.ds(col, cb)])
                return carry

            lax.fori_loop(0, nrit, wbody, 0)
            plsc.subcore_barrier()

    return k(eprime, dst_idx, zrows)


def kernel(mesh2grid_edge_features, grid_node_features, mesh_node_features,
           edge_index, We1, be1, We2, be2, ge_scale, ge_bias,
           Wn1, bn1, Wn2, bn2, gn_scale, gn_bias):
    h = mesh2grid_edge_features.shape[1]
    n = grid_node_features.shape[0]
    dst = edge_index[:, 0].astype(jnp.int32)
    src = edge_index[:, 1].astype(jnp.int32)

    w1e, w1s, w1d = We1[:h], We1[h:2 * h], We1[2 * h:]
    ps, pd = _tc_preproj(mesh_node_features, grid_node_features, w1s, w1d, be1)
    ga, gb = _sc_gather(ps, pd, src, dst)
    ep = _tc_edge(mesh2grid_edge_features, ga, gb, w1e, We2, be2,
                  ge_scale, ge_bias)
    zrows = jnp.zeros((n, 128), jnp.float32)
    agg = _sc_scatter(ep, dst, zrows)
    return _tc_node(grid_node_features, agg, Wn1[:h], Wn1[h:], bn1,
                    Wn2, bn2, gn_scale, gn_bias)
